# baseline (speedup 1.0000x reference)
import jax
import jax.numpy as jnp
from jax.experimental import pallas as pl
from jax.experimental.pallas import tpu as pltpu

SPLIT_INTO = 7
PERM = (2, 4, 3, 6, 5, 0, 1)


def kernel(x):
    b, t, f = x.shape
    chunk = t // SPLIT_INTO

    def body(x_hbm, out_hbm, sem):
        copies = [
            pltpu.make_async_copy(
                x_hbm.at[:, pl.ds(PERM[c] * chunk, chunk), :],
                out_hbm.at[:, pl.ds(c * chunk, chunk), :],
                sem.at[c],
            )
            for c in range(SPLIT_INTO)
        ]
        for cp in copies:
            cp.start()
        for cp in copies:
            cp.wait()

    return pl.pallas_call(
        body,
        out_shape=jax.ShapeDtypeStruct((b, t, f), jnp.float32),
        in_specs=[pl.BlockSpec(memory_space=pl.ANY)],
        out_specs=pl.BlockSpec(memory_space=pl.ANY),
        scratch_shapes=[pltpu.SemaphoreType.DMA((SPLIT_INTO,))],
    )(x)


# speedup vs baseline: 19.3596x; 19.3596x over previous
import jax
import jax.numpy as jnp
from jax.experimental import pallas as pl
from jax.experimental.pallas import tpu as pltpu

SPLIT_INTO = 7
PERM = (2, 4, 3, 6, 5, 0, 1)
NBUF = 4
AHEAD = 2
SLAB = 8


def kernel(x):
    b, t, f = x.shape
    chunk = t // SPLIT_INTO
    nslab = b // SLAB
    n = SPLIT_INTO * nslab

    def body(x_hbm, out_hbm, buf, sem_in, sem_out):
        def start_in(j):
            c, s = divmod(j, nslab)
            return pltpu.make_async_copy(
                x_hbm.at[pl.ds(s * SLAB, SLAB), pl.ds(PERM[c] * chunk, chunk), :],
                buf.at[j % NBUF],
                sem_in.at[j % NBUF],
            )

        def start_out(j):
            c, s = divmod(j, nslab)
            return pltpu.make_async_copy(
                buf.at[j % NBUF],
                out_hbm.at[pl.ds(s * SLAB, SLAB), pl.ds(c * chunk, chunk), :],
                sem_out.at[j % NBUF],
            )

        ins, outs = {}, {}
        for j in range(AHEAD):
            ins[j] = start_in(j)
            ins[j].start()
        for j in range(n):
            k = j + AHEAD
            if k < n:
                if k >= NBUF:
                    outs[k - NBUF].wait()
                ins[k] = start_in(k)
                ins[k].start()
            ins[j].wait()
            outs[j] = start_out(j)
            outs[j].start()
        for j in range(n - NBUF, n):
            outs[j].wait()

    return pl.pallas_call(
        body,
        out_shape=jax.ShapeDtypeStruct((b, t, f), jnp.float32),
        in_specs=[pl.BlockSpec(memory_space=pl.ANY)],
        out_specs=pl.BlockSpec(memory_space=pl.ANY),
        scratch_shapes=[
            pltpu.VMEM((NBUF, SLAB, chunk, f), jnp.float32),
            pltpu.SemaphoreType.DMA((NBUF,)),
            pltpu.SemaphoreType.DMA((NBUF,)),
        ],
    )(x)


# speedup vs baseline: 76.8748x; 3.9709x over previous
import jax
import jax.numpy as jnp
from jax.experimental import pallas as pl
from jax.experimental.pallas import tpu as pltpu

SPLIT_INTO = 7
PERM = (2, 4, 3, 6, 5, 0, 1)
LANES = 128
NBUF = 4
AHEAD = 2
SLAB = 32


def kernel(x):
    b, t, f = x.shape
    rows = t * f // LANES
    crows = rows // SPLIT_INTO
    nslab = b // SLAB
    n = SPLIT_INTO * nslab

    def body(x_hbm, out_hbm, buf, sem_in, sem_out):
        def start_in(j):
            c, s = divmod(j, nslab)
            return pltpu.make_async_copy(
                x_hbm.at[pl.ds(s * SLAB, SLAB), pl.ds(PERM[c] * crows, crows), :],
                buf.at[j % NBUF],
                sem_in.at[j % NBUF],
            )

        def start_out(j):
            c, s = divmod(j, nslab)
            return pltpu.make_async_copy(
                buf.at[j % NBUF],
                out_hbm.at[pl.ds(s * SLAB, SLAB), pl.ds(c * crows, crows), :],
                sem_out.at[j % NBUF],
            )

        ins, outs = {}, {}
        for j in range(AHEAD):
            ins[j] = start_in(j)
            ins[j].start()
        for j in range(n):
            k = j + AHEAD
            if k < n:
                if k >= NBUF:
                    outs[k - NBUF].wait()
                ins[k] = start_in(k)
                ins[k].start()
            ins[j].wait()
            outs[j] = start_out(j)
            outs[j].start()
        for j in range(n - NBUF, n):
            outs[j].wait()

    xv = x.reshape(b, rows, LANES)
    out = pl.pallas_call(
        body,
        out_shape=jax.ShapeDtypeStruct((b, rows, LANES), jnp.float32),
        in_specs=[pl.BlockSpec(memory_space=pl.ANY)],
        out_specs=pl.BlockSpec(memory_space=pl.ANY),
        scratch_shapes=[
            pltpu.VMEM((NBUF, SLAB, crows, LANES), jnp.float32),
            pltpu.SemaphoreType.DMA((NBUF,)),
            pltpu.SemaphoreType.DMA((NBUF,)),
        ],
    )(xv)
    return out.reshape(b, t, f)
